# initial kernel scaffold (unmeasured)
import functools

import jax
import jax.numpy as jnp
from jax import lax
from jax.experimental import pallas as pl
from jax.experimental.pallas import tpu as pltpu

N_DEV = 8
HQ = 8
DH = 128
SQ = 256
SKV = 4096
QBLK = 64
SCALE = 0.08838834764831843
NEG = -1e9

RING = (0, 1, 2, 3, 7, 6, 5, 4)
NEXT = (1, 2, 3, 7, 0, 4, 5, 6)
PREV = (4, 0, 1, 2, 5, 6, 7, 3)
POS = (0, 1, 2, 3, 7, 6, 5, 4)


def _body(meta_ref, q_ref, k_ref, v_ref, out_ref,
          qs_ref, acc_ref, st_ref,
          qsend, qrecv, asend, arecv, ssend, srecv):
    right = meta_ref[8]
    left = meta_ref[9]
    my = meta_ref[10]

    barrier = pltpu.get_barrier_semaphore()
    for nbr in (left, right):
        pl.semaphore_signal(barrier, inc=1, device_id=(nbr,),
                            device_id_type=pl.DeviceIdType.MESH)
    pl.semaphore_wait(barrier, 2)

    col_blk = lax.broadcasted_iota(jnp.int32, (SQ, SKV), 1) // QBLK
    row_blk = lax.broadcasted_iota(jnp.int32, (SQ, SKV), 0) // QBLK
    kb = my * (SKV // QBLK) + col_blk

    def _mk(src, dst, ssem, rsem, dev):
        return pltpu.make_async_remote_copy(
            src_ref=src, dst_ref=dst, send_sem=ssem, recv_sem=rsem,
            device_id=(dev,), device_id_type=pl.DeviceIdType.MESH)

    sends = []
    for s in range(N_DEV):
        if s > 0:
            _mk(qs_ref.at[s], qs_ref.at[s], qsend.at[s], qrecv.at[s],
                left).wait_recv()
            _mk(acc_ref.at[s], acc_ref.at[s], asend.at[s], arecv.at[s],
                left).wait_recv()
            _mk(st_ref.at[s], st_ref.at[s], ssend.at[s], srecv.at[s],
                left).wait_recv()

        c = meta_ref[s]
        qb = c * (SQ // QBLK) + row_blk
        mask = (qb == kb) | (kb == 0) | ((qb + kb) % 3 == 0)
        bias = jnp.where(mask, 0.0, NEG).astype(jnp.float32)

        for h in range(HQ):
            q = q_ref[h] if s == 0 else qs_ref[s, h]
            scores = lax.dot_general(
                q, k_ref[h], (((1,), (1,)), ((), ())),
                preferred_element_type=jnp.float32)
            scores = scores * SCALE + bias
            row_max = jnp.max(scores, axis=1, keepdims=True)
            if s == 0:
                m_new = row_max
                p = jnp.exp(scores - m_new)
                l_new = jnp.sum(p, axis=1, keepdims=True)
                acc_new = lax.dot_general(
                    p.astype(jnp.bfloat16), v_ref[h],
                    (((1,), (0,)), ((), ())),
                    preferred_element_type=jnp.float32)
            else:
                m_old = st_ref[s, 0, :, h:h + 1]
                l_old = st_ref[s, 1, :, h:h + 1]
                m_new = jnp.maximum(m_old, row_max)
                alpha = jnp.exp(m_old - m_new)
                p = jnp.exp(scores - m_new)
                l_new = l_old * alpha + jnp.sum(p, axis=1, keepdims=True)
                acc_new = acc_ref[s, h] * alpha + lax.dot_general(
                    p.astype(jnp.bfloat16), v_ref[h],
                    (((1,), (0,)), ((), ())),
                    preferred_element_type=jnp.float32)
            acc_ref[s, h] = acc_new
            st_ref[s, 0, :, h:h + 1] = m_new
            st_ref[s, 1, :, h:h + 1] = l_new

        nxt = (s + 1) % N_DEV
        if s < N_DEV - 1:
            qsrc = q_ref if s == 0 else qs_ref.at[s]
            r = _mk(qsrc, qs_ref.at[nxt], qsend.at[s], qrecv.at[nxt], right)
            r.start()
            sends.append(r)
        r = _mk(acc_ref.at[s], acc_ref.at[nxt], asend.at[s], arecv.at[nxt],
                right)
        r.start()
        sends.append(r)
        r = _mk(st_ref.at[s], st_ref.at[nxt], ssend.at[s], srecv.at[nxt],
                right)
        r.start()
        sends.append(r)

    _mk(acc_ref.at[0], acc_ref.at[0], asend.at[0], arecv.at[0],
        left).wait_recv()
    _mk(st_ref.at[0], st_ref.at[0], ssend.at[0], srecv.at[0],
        left).wait_recv()

    for h in range(HQ):
        out_ref[h] = acc_ref[0, h] / st_ref[0, 1, :, h:h + 1]

    for r in sends:
        r.wait_send()

    @functools.partial(pl.run_scoped, exit_sem=pltpu.SemaphoreType.REGULAR)
    def _(exit_sem):
        for nbr in (left, right):
            pl.semaphore_signal(exit_sem, inc=1, device_id=(nbr,),
                                device_id_type=pl.DeviceIdType.MESH)
        pl.semaphore_wait(exit_sem, 2)


def kernel(x, Wq, K_ext, V_ext, Wo):
    q = (x[0] @ Wq).reshape(SQ, HQ, DH).transpose(1, 0, 2)
    q = q.astype(jnp.bfloat16)
    k = K_ext[0].transpose(1, 0, 2).astype(jnp.bfloat16)
    v = V_ext[0].transpose(1, 0, 2).astype(jnp.bfloat16)

    my = lax.axis_index("i")
    ring = jnp.asarray(RING, jnp.int32)
    pos = jnp.asarray(POS, jnp.int32)[my]
    owners = ring[(pos - jnp.arange(N_DEV, dtype=jnp.int32)) % N_DEV]
    meta = jnp.concatenate([
        owners,
        jnp.asarray(NEXT, jnp.int32)[my][None],
        jnp.asarray(PREV, jnp.int32)[my][None],
        my.astype(jnp.int32)[None],
    ])

    ctx = pl.pallas_call(
        _body,
        out_shape=jax.ShapeDtypeStruct((HQ, SQ, DH), jnp.float32),
        in_specs=[
            pl.BlockSpec(memory_space=pltpu.SMEM),
            pl.BlockSpec(memory_space=pltpu.VMEM),
            pl.BlockSpec(memory_space=pltpu.VMEM),
            pl.BlockSpec(memory_space=pltpu.VMEM),
        ],
        out_specs=pl.BlockSpec(memory_space=pltpu.VMEM),
        scratch_shapes=[
            pltpu.VMEM((N_DEV, HQ, SQ, DH), jnp.bfloat16),
            pltpu.VMEM((N_DEV, HQ, SQ, DH), jnp.float32),
            pltpu.VMEM((N_DEV, 2, SQ, HQ), jnp.float32),
            pltpu.SemaphoreType.DMA((N_DEV,)),
            pltpu.SemaphoreType.DMA((N_DEV,)),
            pltpu.SemaphoreType.DMA((N_DEV,)),
            pltpu.SemaphoreType.DMA((N_DEV,)),
            pltpu.SemaphoreType.DMA((N_DEV,)),
            pltpu.SemaphoreType.DMA((N_DEV,)),
        ],
        compiler_params=pltpu.CompilerParams(collective_id=0),
    )(meta, q, k, v)

    out = ctx.transpose(1, 0, 2).reshape(SQ, HQ * DH) @ Wo
    return out[None]


# baseline (device time: 396893 ns/iter reference)
import functools

import jax
import jax.numpy as jnp
from jax import lax
from jax.experimental import pallas as pl
from jax.experimental.pallas import tpu as pltpu

N_DEV = 8
HQ = 8
DH = 128
SQ = 256
SKV = 4096
QBLK = 64
SCALE = 0.08838834764831843
NEG = -1e9

RING = (0, 1, 2, 3, 7, 6, 5, 4)
NEXT = (1, 2, 3, 7, 0, 4, 5, 6)
PREV = (4, 0, 1, 2, 5, 6, 7, 3)
POS = (0, 1, 2, 3, 7, 6, 5, 4)


def _body(meta_ref, q_ref, k_ref, v_ref, out_ref,
          qs_ref, acc_ref, st_ref, kbuf, vbuf,
          qsend, qrecv, asend, arecv, ssend, srecv, ksem, vsem):
    right = meta_ref[8]
    left = meta_ref[9]
    my = meta_ref[10]

    qs_ref[0] = q_ref[...]
    acc_ref[0] = jnp.zeros((HQ, SQ, DH), jnp.float32)
    st_ref[0, 0] = jnp.full((SQ, HQ), -1e30, jnp.float32)
    st_ref[0, 1] = jnp.zeros((SQ, HQ), jnp.float32)

    barrier = pltpu.get_barrier_semaphore()
    for nbr in (left, right):
        pl.semaphore_signal(barrier, inc=1, device_id=(nbr,),
                            device_id_type=pl.DeviceIdType.MESH)
    pl.semaphore_wait(barrier, 2)

    row_blk = lax.broadcasted_iota(jnp.int32, (SQ, 1), 0) // QBLK
    col_blk = lax.broadcasted_iota(jnp.int32, (1, SKV), 1) // QBLK
    kb = my * (SKV // QBLK) + col_blk

    def _mk(src, dst, ssem, rsem, dev):
        return pltpu.make_async_remote_copy(
            src_ref=src, dst_ref=dst, send_sem=ssem, recv_sem=rsem,
            device_id=(dev,), device_id_type=pl.DeviceIdType.MESH)

    def _hop(s, nxt, sem_slot):
        return (
            _mk(qs_ref.at[s], qs_ref.at[nxt], qsend.at[sem_slot],
                qrecv.at[nxt], right),
            _mk(acc_ref.at[s], acc_ref.at[nxt], asend.at[sem_slot],
                arecv.at[nxt], right),
            _mk(st_ref.at[s], st_ref.at[nxt], ssend.at[sem_slot],
                srecv.at[nxt], right),
        )

    def step(s, carry):
        @pl.when(s > 0)
        def _():
            for r in _hop(s - 1, s, s - 1):
                r.wait_send()
            for r in _hop(s, s, s):
                r.wait_recv()

        pltpu.make_async_copy(k_ref.at[0], kbuf.at[0], ksem.at[0]).start()
        pltpu.make_async_copy(v_ref.at[0], vbuf.at[0], vsem.at[0]).start()

        c = meta_ref[s]
        qb = c * (SQ // QBLK) + row_blk
        mask = (qb == kb) | (kb == 0) | ((qb + kb) % 3 == 0)
        bias = jnp.where(mask, 0.0, NEG).astype(jnp.float32)

        for h in range(HQ):
            b = h % 2
            pltpu.make_async_copy(k_ref.at[h], kbuf.at[b], ksem.at[b]).wait()
            pltpu.make_async_copy(v_ref.at[h], vbuf.at[b], vsem.at[b]).wait()
            if h + 1 < HQ:
                nb = (h + 1) % 2
                pltpu.make_async_copy(
                    k_ref.at[h + 1], kbuf.at[nb], ksem.at[nb]).start()
                pltpu.make_async_copy(
                    v_ref.at[h + 1], vbuf.at[nb], vsem.at[nb]).start()
            scores = lax.dot_general(
                qs_ref[s, h], kbuf[b], (((1,), (1,)), ((), ())),
                preferred_element_type=jnp.float32)
            scores = scores * SCALE + bias
            row_max = jnp.max(scores, axis=1, keepdims=True)
            m_old = st_ref[s, 0, :, h:h + 1]
            l_old = st_ref[s, 1, :, h:h + 1]
            m_new = jnp.maximum(m_old, row_max)
            alpha = jnp.exp(m_old - m_new)
            p = jnp.exp(scores - m_new)
            l_new = l_old * alpha + jnp.sum(p, axis=1, keepdims=True)
            acc_new = acc_ref[s, h] * alpha + lax.dot_general(
                p.astype(jnp.bfloat16), vbuf[b], (((1,), (0,)), ((), ())),
                preferred_element_type=jnp.float32)
            acc_ref[s, h] = acc_new
            st_ref[s, 0, :, h:h + 1] = m_new
            st_ref[s, 1, :, h:h + 1] = l_new

        for r in _hop(s, (s + 1) % N_DEV, s):
            r.start()
        return carry

    lax.fori_loop(0, N_DEV, step, 0)

    for r in _hop(N_DEV - 1, 0, N_DEV - 1):
        r.wait_send()
    for r in _hop(0, 0, 0):
        r.wait_recv()

    for h in range(HQ):
        out_ref[h] = acc_ref[0, h] / st_ref[0, 1, :, h:h + 1]

    @functools.partial(pl.run_scoped, exit_sem=pltpu.SemaphoreType.REGULAR)
    def _(exit_sem):
        for nbr in (left, right):
            pl.semaphore_signal(exit_sem, inc=1, device_id=(nbr,),
                                device_id_type=pl.DeviceIdType.MESH)
        pl.semaphore_wait(exit_sem, 2)


def kernel(x, Wq, K_ext, V_ext, Wo):
    q = (x[0] @ Wq).reshape(SQ, HQ, DH).transpose(1, 0, 2)
    q = q.astype(jnp.bfloat16)
    k = K_ext[0].transpose(1, 0, 2).astype(jnp.bfloat16)
    v = V_ext[0].transpose(1, 0, 2).astype(jnp.bfloat16)

    my = lax.axis_index("i")
    ring = jnp.asarray(RING, jnp.int32)
    pos = jnp.asarray(POS, jnp.int32)[my]
    owners = ring[(pos - jnp.arange(N_DEV, dtype=jnp.int32)) % N_DEV]
    meta = jnp.concatenate([
        owners,
        jnp.asarray(NEXT, jnp.int32)[my][None],
        jnp.asarray(PREV, jnp.int32)[my][None],
        my.astype(jnp.int32)[None],
    ])

    ctx = pl.pallas_call(
        _body,
        out_shape=jax.ShapeDtypeStruct((HQ, SQ, DH), jnp.float32),
        in_specs=[
            pl.BlockSpec(memory_space=pltpu.SMEM),
            pl.BlockSpec(memory_space=pltpu.VMEM),
            pl.BlockSpec(memory_space=pltpu.MemorySpace.HBM),
            pl.BlockSpec(memory_space=pltpu.MemorySpace.HBM),
        ],
        out_specs=pl.BlockSpec(memory_space=pltpu.VMEM),
        scratch_shapes=[
            pltpu.VMEM((N_DEV, HQ, SQ, DH), jnp.bfloat16),
            pltpu.VMEM((N_DEV, HQ, SQ, DH), jnp.float32),
            pltpu.VMEM((N_DEV, 2, SQ, HQ), jnp.float32),
            pltpu.VMEM((2, SKV, DH), jnp.bfloat16),
            pltpu.VMEM((2, SKV, DH), jnp.bfloat16),
            pltpu.SemaphoreType.DMA((N_DEV,)),
            pltpu.SemaphoreType.DMA((N_DEV,)),
            pltpu.SemaphoreType.DMA((N_DEV,)),
            pltpu.SemaphoreType.DMA((N_DEV,)),
            pltpu.SemaphoreType.DMA((N_DEV,)),
            pltpu.SemaphoreType.DMA((N_DEV,)),
            pltpu.SemaphoreType.DMA((2,)),
            pltpu.SemaphoreType.DMA((2,)),
        ],
        compiler_params=pltpu.CompilerParams(
            collective_id=0,
            vmem_limit_bytes=58 * 1024 * 1024,
        ),
    )(meta, q, k, v)

    out = ctx.transpose(1, 0, 2).reshape(SQ, HQ * DH) @ Wo
    return out[None]


# device time: 293480 ns/iter; 1.3524x vs baseline; 1.3524x over previous
import functools

import jax
import jax.numpy as jnp
from jax import lax
from jax.experimental import pallas as pl
from jax.experimental.pallas import tpu as pltpu

N_DEV = 8
HQ = 8
DH = 128
SQ = 256
SKV = 4096
QBLK = 64
SCALE = 0.08838834764831843
NEG = -1e9

RING = (0, 1, 2, 3, 7, 6, 5, 4)
NEXT = (1, 2, 3, 7, 0, 4, 5, 6)
PREV = (4, 0, 1, 2, 5, 6, 7, 3)
POS = (0, 1, 2, 3, 7, 6, 5, 4)


def _body(meta_ref, q_ref, k_ref, v_ref, out_ref,
          qs_ref, acc_ref, st_ref, pv_ref, stl_ref, kbuf, vbuf,
          qsend, qrecv, asend, arecv, ssend, srecv, ksem, vsem):
    right = meta_ref[8]
    left = meta_ref[9]
    my = meta_ref[10]

    qs_ref[0] = q_ref[...]
    acc_ref[0] = jnp.zeros((HQ, SQ, DH), jnp.float32)
    st_ref[0, 0] = jnp.full((SQ, HQ), -1e30, jnp.float32)
    st_ref[0, 1] = jnp.zeros((SQ, HQ), jnp.float32)

    barrier = pltpu.get_barrier_semaphore()
    for nbr in (left, right):
        pl.semaphore_signal(barrier, inc=1, device_id=(nbr,),
                            device_id_type=pl.DeviceIdType.MESH)
    pl.semaphore_wait(barrier, 2)

    row_blk = lax.broadcasted_iota(jnp.int32, (SQ, 1), 0) // QBLK
    col_blk = lax.broadcasted_iota(jnp.int32, (1, SKV), 1) // QBLK
    kb = my * (SKV // QBLK) + col_blk

    def _mk(src, dst, ssem, rsem, dev):
        return pltpu.make_async_remote_copy(
            src_ref=src, dst_ref=dst, send_sem=ssem, recv_sem=rsem,
            device_id=(dev,), device_id_type=pl.DeviceIdType.MESH)

    def _q_hop(s, nxt):
        return _mk(qs_ref.at[s], qs_ref.at[nxt], qsend.at[s],
                   qrecv.at[nxt], right)

    def _acc_hop(s, nxt):
        return (_mk(acc_ref.at[s], acc_ref.at[nxt], asend.at[s],
                    arecv.at[nxt], right),
                _mk(st_ref.at[s], st_ref.at[nxt], ssend.at[s],
                    srecv.at[nxt], right))

    def step(s, carry):
        nxt = (s + 1) % N_DEV

        @pl.when(s > 0)
        def _():
            _q_hop(s - 1, s).wait_send()
            _q_hop(s, s).wait_recv()

        pltpu.make_async_copy(k_ref.at[0], kbuf.at[0], ksem.at[0]).start()
        pltpu.make_async_copy(v_ref.at[0], vbuf.at[0], vsem.at[0]).start()

        c = meta_ref[s]
        qb = c * (SQ // QBLK) + row_blk
        mask = (qb == kb) | (kb == 0) | ((qb + kb) % 3 == 0)
        bias = jnp.where(mask, 0.0, NEG).astype(jnp.float32)

        for h in range(HQ):
            b = h % 2
            pltpu.make_async_copy(k_ref.at[h], kbuf.at[b], ksem.at[b]).wait()
            pltpu.make_async_copy(v_ref.at[h], vbuf.at[b], vsem.at[b]).wait()
            if h + 1 < HQ:
                nb = (h + 1) % 2
                pltpu.make_async_copy(
                    k_ref.at[h + 1], kbuf.at[nb], ksem.at[nb]).start()
                pltpu.make_async_copy(
                    v_ref.at[h + 1], vbuf.at[nb], vsem.at[nb]).start()
            scores = lax.dot_general(
                qs_ref[s, h], kbuf[b], (((1,), (1,)), ((), ())),
                preferred_element_type=jnp.float32)
            scores = scores * SCALE + bias
            m_loc = jnp.max(scores, axis=1, keepdims=True)
            p = jnp.exp(scores - m_loc)
            stl_ref[0, :, h:h + 1] = m_loc
            stl_ref[1, :, h:h + 1] = jnp.sum(p, axis=1, keepdims=True)
            pv_ref[h] = lax.dot_general(
                p.astype(jnp.bfloat16), vbuf[b], (((1,), (0,)), ((), ())),
                preferred_element_type=jnp.float32).astype(jnp.bfloat16)

        @pl.when(s > 0)
        def _():
            for r in _acc_hop(s - 1, s):
                r.wait_send()
            for r in _acc_hop(s, s):
                r.wait_recv()

        for h in range(HQ):
            m_in = st_ref[s, 0, :, h:h + 1]
            l_in = st_ref[s, 1, :, h:h + 1]
            m_loc = stl_ref[0, :, h:h + 1]
            l_loc = stl_ref[1, :, h:h + 1]
            m_new = jnp.maximum(m_in, m_loc)
            a_in = jnp.exp(m_in - m_new)
            a_loc = jnp.exp(m_loc - m_new)
            st_ref[s, 0, :, h:h + 1] = m_new
            st_ref[s, 1, :, h:h + 1] = l_in * a_in + l_loc * a_loc
            acc_ref[s, h] = (acc_ref[s, h] * a_in
                             + pv_ref[h].astype(jnp.float32) * a_loc)

        @pl.when(s < N_DEV - 1)
        def _():
            _q_hop(s, nxt).start()
        for r in _acc_hop(s, nxt):
            r.start()
        return carry

    lax.fori_loop(0, N_DEV, step, 0)

    for r in _acc_hop(N_DEV - 1, 0):
        r.wait_send()
    for r in _acc_hop(0, 0):
        r.wait_recv()

    for h in range(HQ):
        out_ref[h] = acc_ref[0, h] / st_ref[0, 1, :, h:h + 1]

    @functools.partial(pl.run_scoped, exit_sem=pltpu.SemaphoreType.REGULAR)
    def _(exit_sem):
        for nbr in (left, right):
            pl.semaphore_signal(exit_sem, inc=1, device_id=(nbr,),
                                device_id_type=pl.DeviceIdType.MESH)
        pl.semaphore_wait(exit_sem, 2)


def kernel(x, Wq, K_ext, V_ext, Wo):
    q = (x[0] @ Wq).reshape(SQ, HQ, DH).transpose(1, 0, 2)
    q = q.astype(jnp.bfloat16)
    k = K_ext[0].transpose(1, 0, 2).astype(jnp.bfloat16)
    v = V_ext[0].transpose(1, 0, 2).astype(jnp.bfloat16)

    my = lax.axis_index("i")
    ring = jnp.asarray(RING, jnp.int32)
    pos = jnp.asarray(POS, jnp.int32)[my]
    owners = ring[(pos - jnp.arange(N_DEV, dtype=jnp.int32)) % N_DEV]
    meta = jnp.concatenate([
        owners,
        jnp.asarray(NEXT, jnp.int32)[my][None],
        jnp.asarray(PREV, jnp.int32)[my][None],
        my.astype(jnp.int32)[None],
    ])

    ctx = pl.pallas_call(
        _body,
        out_shape=jax.ShapeDtypeStruct((HQ, SQ, DH), jnp.float32),
        in_specs=[
            pl.BlockSpec(memory_space=pltpu.SMEM),
            pl.BlockSpec(memory_space=pltpu.VMEM),
            pl.BlockSpec(memory_space=pltpu.MemorySpace.HBM),
            pl.BlockSpec(memory_space=pltpu.MemorySpace.HBM),
        ],
        out_specs=pl.BlockSpec(memory_space=pltpu.VMEM),
        scratch_shapes=[
            pltpu.VMEM((N_DEV, HQ, SQ, DH), jnp.bfloat16),
            pltpu.VMEM((N_DEV, HQ, SQ, DH), jnp.float32),
            pltpu.VMEM((N_DEV, 2, SQ, HQ), jnp.float32),
            pltpu.VMEM((HQ, SQ, DH), jnp.bfloat16),
            pltpu.VMEM((2, SQ, HQ), jnp.float32),
            pltpu.VMEM((2, SKV, DH), jnp.bfloat16),
            pltpu.VMEM((2, SKV, DH), jnp.bfloat16),
            pltpu.SemaphoreType.DMA((N_DEV,)),
            pltpu.SemaphoreType.DMA((N_DEV,)),
            pltpu.SemaphoreType.DMA((N_DEV,)),
            pltpu.SemaphoreType.DMA((N_DEV,)),
            pltpu.SemaphoreType.DMA((N_DEV,)),
            pltpu.SemaphoreType.DMA((N_DEV,)),
            pltpu.SemaphoreType.DMA((2,)),
            pltpu.SemaphoreType.DMA((2,)),
        ],
        compiler_params=pltpu.CompilerParams(
            collective_id=0,
            vmem_limit_bytes=58 * 1024 * 1024,
        ),
    )(meta, q, k, v)

    out = ctx.transpose(1, 0, 2).reshape(SQ, HQ * DH) @ Wo
    return out[None]


# device time: 241447 ns/iter; 1.6438x vs baseline; 1.2155x over previous
import functools

import jax
import jax.numpy as jnp
from jax import lax
from jax.experimental import pallas as pl
from jax.experimental.pallas import tpu as pltpu

N_DEV = 8
HQ = 8
DH = 128
SQ = 256
SKV = 4096
QBLK = 64
SCALE = 0.08838834764831843
NEG = -1e9

RING = (0, 1, 2, 3, 7, 6, 5, 4)
NEXT = (1, 2, 3, 7, 0, 4, 5, 6)
PREV = (4, 0, 1, 2, 5, 6, 7, 3)
POS = (0, 1, 2, 3, 7, 6, 5, 4)


def _body(meta_ref, q_ref, k_ref, v_ref, out_ref,
          qs_ref, acc_ref, st_ref, pv_ref, stl_ref, kbuf, vbuf,
          qsend, qrecv, asend, arecv, ssend, srecv, ksem, vsem):
    right = meta_ref[8]
    left = meta_ref[9]
    my = meta_ref[10]

    qs_ref[0] = q_ref[...]
    acc_ref[0] = jnp.zeros((HQ, SQ, DH), jnp.bfloat16)
    st_ref[0, 0] = jnp.full((SQ, HQ), -1e30, jnp.float32)
    st_ref[0, 1] = jnp.zeros((SQ, HQ), jnp.float32)

    barrier = pltpu.get_barrier_semaphore()
    for nbr in (left, right):
        pl.semaphore_signal(barrier, inc=1, device_id=(nbr,),
                            device_id_type=pl.DeviceIdType.MESH)
    pl.semaphore_wait(barrier, 2)

    row_blk = lax.broadcasted_iota(jnp.int32, (SQ, 1), 0) // QBLK
    col_blk = lax.broadcasted_iota(jnp.int32, (1, SKV), 1) // QBLK
    kb = my * (SKV // QBLK) + col_blk

    def _mk(src, dst, ssem, rsem, dev):
        return pltpu.make_async_remote_copy(
            src_ref=src, dst_ref=dst, send_sem=ssem, recv_sem=rsem,
            device_id=(dev,), device_id_type=pl.DeviceIdType.MESH)

    def _q_hop(s, nxt):
        return _mk(qs_ref.at[s], qs_ref.at[nxt], qsend.at[s],
                   qrecv.at[nxt], right)

    def _acc_hop(s, nxt):
        return (_mk(acc_ref.at[s], acc_ref.at[nxt], asend.at[s],
                    arecv.at[nxt], right),
                _mk(st_ref.at[s], st_ref.at[nxt], ssend.at[s],
                    srecv.at[nxt], right))

    def step(s, carry):
        nxt = (s + 1) % N_DEV

        @pl.when(s > 0)
        def _():
            _q_hop(s - 1, s).wait_send()
            _q_hop(s, s).wait_recv()

        @pl.when(s < N_DEV - 1)
        def _():
            _q_hop(s, nxt).start()

        pltpu.make_async_copy(k_ref.at[0], kbuf.at[0], ksem.at[0]).start()
        pltpu.make_async_copy(v_ref.at[0], vbuf.at[0], vsem.at[0]).start()

        c = meta_ref[s]
        qb = c * (SQ // QBLK) + row_blk
        mask = (qb == kb) | (kb == 0) | ((qb + kb) % 3 == 0)
        bias = jnp.where(mask, 0.0, NEG).astype(jnp.float32)

        for h in range(HQ):
            b = h % 2
            pltpu.make_async_copy(k_ref.at[h], kbuf.at[b], ksem.at[b]).wait()
            pltpu.make_async_copy(v_ref.at[h], vbuf.at[b], vsem.at[b]).wait()
            if h + 1 < HQ:
                nb = (h + 1) % 2
                pltpu.make_async_copy(
                    k_ref.at[h + 1], kbuf.at[nb], ksem.at[nb]).start()
                pltpu.make_async_copy(
                    v_ref.at[h + 1], vbuf.at[nb], vsem.at[nb]).start()
            scores = lax.dot_general(
                qs_ref[s, h], kbuf[b], (((1,), (1,)), ((), ())),
                preferred_element_type=jnp.float32)
            scores = scores * SCALE + bias
            m_loc = jnp.max(scores, axis=1, keepdims=True)
            p = jnp.exp(scores - m_loc)
            stl_ref[0, :, h:h + 1] = m_loc
            stl_ref[1, :, h:h + 1] = jnp.sum(p, axis=1, keepdims=True)
            pv_ref[h] = lax.dot_general(
                p.astype(jnp.bfloat16), vbuf[b], (((1,), (0,)), ((), ())),
                preferred_element_type=jnp.float32).astype(jnp.bfloat16)

        @pl.when(s > 0)
        def _():
            for r in _acc_hop(s - 1, s):
                r.wait_send()
            for r in _acc_hop(s, s):
                r.wait_recv()

        for h in range(HQ):
            m_in = st_ref[s, 0, :, h:h + 1]
            l_in = st_ref[s, 1, :, h:h + 1]
            m_loc = stl_ref[0, :, h:h + 1]
            l_loc = stl_ref[1, :, h:h + 1]
            m_new = jnp.maximum(m_in, m_loc)
            a_in = jnp.exp(m_in - m_new)
            a_loc = jnp.exp(m_loc - m_new)
            st_ref[s, 0, :, h:h + 1] = m_new
            st_ref[s, 1, :, h:h + 1] = l_in * a_in + l_loc * a_loc
            acc_ref[s, h] = (
                acc_ref[s, h].astype(jnp.float32) * a_in
                + pv_ref[h].astype(jnp.float32) * a_loc
            ).astype(jnp.bfloat16)

        for r in _acc_hop(s, nxt):
            r.start()
        return carry

    lax.fori_loop(0, N_DEV, step, 0)

    for r in _acc_hop(N_DEV - 1, 0):
        r.wait_send()
    for r in _acc_hop(0, 0):
        r.wait_recv()

    for h in range(HQ):
        out_ref[h] = (acc_ref[0, h].astype(jnp.float32)
                      / st_ref[0, 1, :, h:h + 1])

    @functools.partial(pl.run_scoped, exit_sem=pltpu.SemaphoreType.REGULAR)
    def _(exit_sem):
        for nbr in (left, right):
            pl.semaphore_signal(exit_sem, inc=1, device_id=(nbr,),
                                device_id_type=pl.DeviceIdType.MESH)
        pl.semaphore_wait(exit_sem, 2)


def kernel(x, Wq, K_ext, V_ext, Wo):
    q = (x[0] @ Wq).reshape(SQ, HQ, DH).transpose(1, 0, 2)
    q = q.astype(jnp.bfloat16)
    k = K_ext[0].transpose(1, 0, 2).astype(jnp.bfloat16)
    v = V_ext[0].transpose(1, 0, 2).astype(jnp.bfloat16)

    my = lax.axis_index("i")
    ring = jnp.asarray(RING, jnp.int32)
    pos = jnp.asarray(POS, jnp.int32)[my]
    owners = ring[(pos - jnp.arange(N_DEV, dtype=jnp.int32)) % N_DEV]
    meta = jnp.concatenate([
        owners,
        jnp.asarray(NEXT, jnp.int32)[my][None],
        jnp.asarray(PREV, jnp.int32)[my][None],
        my.astype(jnp.int32)[None],
    ])

    ctx = pl.pallas_call(
        _body,
        out_shape=jax.ShapeDtypeStruct((HQ, SQ, DH), jnp.float32),
        in_specs=[
            pl.BlockSpec(memory_space=pltpu.SMEM),
            pl.BlockSpec(memory_space=pltpu.VMEM),
            pl.BlockSpec(memory_space=pltpu.MemorySpace.HBM),
            pl.BlockSpec(memory_space=pltpu.MemorySpace.HBM),
        ],
        out_specs=pl.BlockSpec(memory_space=pltpu.VMEM),
        scratch_shapes=[
            pltpu.VMEM((N_DEV, HQ, SQ, DH), jnp.bfloat16),
            pltpu.VMEM((N_DEV, HQ, SQ, DH), jnp.bfloat16),
            pltpu.VMEM((N_DEV, 2, SQ, HQ), jnp.float32),
            pltpu.VMEM((HQ, SQ, DH), jnp.bfloat16),
            pltpu.VMEM((2, SQ, HQ), jnp.float32),
            pltpu.VMEM((2, SKV, DH), jnp.bfloat16),
            pltpu.VMEM((2, SKV, DH), jnp.bfloat16),
            pltpu.SemaphoreType.DMA((N_DEV,)),
            pltpu.SemaphoreType.DMA((N_DEV,)),
            pltpu.SemaphoreType.DMA((N_DEV,)),
            pltpu.SemaphoreType.DMA((N_DEV,)),
            pltpu.SemaphoreType.DMA((N_DEV,)),
            pltpu.SemaphoreType.DMA((N_DEV,)),
            pltpu.SemaphoreType.DMA((2,)),
            pltpu.SemaphoreType.DMA((2,)),
        ],
        compiler_params=pltpu.CompilerParams(
            collective_id=0,
            vmem_limit_bytes=58 * 1024 * 1024,
        ),
    )(meta, q, k, v)

    out = ctx.transpose(1, 0, 2).reshape(SQ, HQ * DH) @ Wo
    return out[None]
